# Initial kernel scaffold; baseline (speedup 1.0000x reference)
import jax
import jax.numpy as jnp
from jax import lax
from jax.experimental import pallas as pl
from jax.experimental.pallas import tpu as pltpu
from jax.experimental.pallas import tpu_sc as plsc

B = 8
H = 512
W = 512
HW = H * W
NC = 2
NS = 16
P = HW // NS
CH = 128
NCH = P // CH
BPC = B // NC
ZB = 8192
RB = 256


def _prep_body(flow_ref, invd_ref, idx_ref, wx_ref, wy_ref, w_ref):
    r = pl.program_id(1)
    fx = flow_ref[0, 0]
    fy = flow_ref[0, 1]
    dv = invd_ref[0, 0]
    y = lax.broadcasted_iota(jnp.float32, (RB, W), 0) + (r * RB).astype(jnp.float32)
    x = lax.broadcasted_iota(jnp.float32, (RB, W), 1)
    tx = jnp.round(x - fx)
    ty = jnp.round(y - fy)
    inr = (tx >= 0.0) & (tx < float(W)) & (ty >= 0.0) & (ty < float(H))
    tgt = tx.astype(jnp.int32) + ty.astype(jnp.int32) * W
    own = (y * float(W) + x).astype(jnp.int32)
    w = jnp.where(inr, dv, 0.0)
    idx_ref[0] = jnp.where(inr, tgt, own)
    wx_ref[0] = fx * w
    wy_ref[0] = fy * w
    w_ref[0] = w


_prep = pl.pallas_call(
    _prep_body,
    grid=(B, H // RB),
    in_specs=[
        pl.BlockSpec((1, 2, RB, W), lambda b, r: (b, 0, r, 0)),
        pl.BlockSpec((1, 1, RB, W), lambda b, r: (b, 0, r, 0)),
    ],
    out_specs=[pl.BlockSpec((1, RB, W), lambda b, r: (b, r, 0))] * 4,
    out_shape=[
        jax.ShapeDtypeStruct((B, H, W), jnp.int32),
        jax.ShapeDtypeStruct((B, H, W), jnp.float32),
        jax.ShapeDtypeStruct((B, H, W), jnp.float32),
        jax.ShapeDtypeStruct((B, H, W), jnp.float32),
    ],
)


def _sc_body(idx_hbm, wx_hbm, wy_hbm, w_hbm, out_hbm,
             idx_v, vx_v, vy_v, vw_v, zb_v, acc_x, acc_y, acc_w):
    c = lax.axis_index("c")
    s = lax.axis_index("s")
    base = s * P

    def _zb(i, carry):
        zb_v[pl.ds(pl.multiple_of(i * 16, 16), 16)] = jnp.zeros((16,), jnp.float32)
        return carry

    lax.fori_loop(0, ZB // 16, _zb, 0)

    for k in range(BPC):
        b = c * BPC + k
        plsc.subcore_barrier()
        for acc in (acc_x, acc_y, acc_w):
            pltpu.sync_copy(zb_v, acc.at[pl.ds(base, ZB)])
            pltpu.sync_copy(zb_v, acc.at[pl.ds(base + ZB, ZB)])
        pltpu.sync_copy(idx_hbm.at[b, s], idx_v)
        pltpu.sync_copy(wx_hbm.at[b, s], vx_v)
        pltpu.sync_copy(wy_hbm.at[b, s], vy_v)
        pltpu.sync_copy(w_hbm.at[b, s], vw_v)
        plsc.subcore_barrier()

        def _scatter(j, carry):
            idx_row = idx_v.at[j]
            off = pl.ds(pl.multiple_of(j * CH, CH), CH)
            pltpu.sync_copy(vx_v.at[off], acc_x.at[idx_row], add=True)
            pltpu.sync_copy(vy_v.at[off], acc_y.at[idx_row], add=True)
            pltpu.sync_copy(vw_v.at[off], acc_w.at[idx_row], add=True)
            return carry

        lax.fori_loop(0, NCH, _scatter, 0)
        plsc.subcore_barrier()

        pltpu.sync_copy(acc_x.at[pl.ds(base, P)], vx_v)
        pltpu.sync_copy(acc_y.at[pl.ds(base, P)], vy_v)
        pltpu.sync_copy(acc_w.at[pl.ds(base, P)], vw_v)

        def _fin(i, carry):
            sl = pl.ds(pl.multiple_of(i * 16, 16), 16)
            ax = vx_v[sl]
            ay = vy_v[sl]
            aw = vw_v[sl]
            inv = jnp.where(ax != 0.0, 1.0 / (aw + 1e-7), 0.0)
            vx_v[sl] = ax * inv
            vy_v[sl] = ay * inv
            return carry

        lax.fori_loop(0, P // 16, _fin, 0)

        pltpu.sync_copy(vx_v, out_hbm.at[b, 0, pl.ds(base, P)])
        pltpu.sync_copy(vy_v, out_hbm.at[b, 1, pl.ds(base, P)])


_scatter_norm = pl.kernel(
    _sc_body,
    out_type=jax.ShapeDtypeStruct((B, 2, HW), jnp.float32),
    mesh=plsc.VectorSubcoreMesh(core_axis_name="c", subcore_axis_name="s"),
    scratch_types=[
        pltpu.VMEM((NCH, CH), jnp.int32),
        pltpu.VMEM((P,), jnp.float32),
        pltpu.VMEM((P,), jnp.float32),
        pltpu.VMEM((P,), jnp.float32),
        pltpu.VMEM((ZB,), jnp.float32),
        pltpu.VMEM_SHARED((HW,), jnp.float32),
        pltpu.VMEM_SHARED((HW,), jnp.float32),
        pltpu.VMEM_SHARED((HW,), jnp.float32),
    ],
)


def kernel(flow, inv_depth):
    idx, wx, wy, w = _prep(flow, inv_depth)
    out = _scatter_norm(
        idx.reshape(B, NS, NCH, CH),
        wx.reshape(B, NS, P),
        wy.reshape(B, NS, P),
        w.reshape(B, NS, P),
    )
    return out.reshape(B, 2, H, W)


# speedup vs baseline: 36.1402x; 36.1402x over previous
import jax
import jax.numpy as jnp
from jax import lax
from jax.experimental import pallas as pl
from jax.experimental.pallas import tpu as pltpu
from jax.experimental.pallas import tpu_sc as plsc

B = 8
H = 512
W = 512
HW = H * W
NC = 2
NS = 16
P = HW // NS
CH = 128
NCH = P // CH
BPC = B // NC
ZB = 8192
RB = 256


def _prep_body(flow_ref, invd_ref, idx_ref, wx_ref, wy_ref, w_ref):
    r = pl.program_id(1)
    fx = flow_ref[0, 0]
    fy = flow_ref[0, 1]
    dv = invd_ref[0, 0]
    y = (lax.broadcasted_iota(jnp.int32, (RB, W), 0) + r * RB).astype(jnp.float32)
    x = lax.broadcasted_iota(jnp.int32, (RB, W), 1).astype(jnp.float32)
    tx = jnp.round(x - fx)
    ty = jnp.round(y - fy)
    inr = (tx >= 0.0) & (tx < float(W)) & (ty >= 0.0) & (ty < float(H))
    tgt = tx.astype(jnp.int32) + ty.astype(jnp.int32) * W
    own = (y * float(W) + x).astype(jnp.int32)
    w = jnp.where(inr, dv, 0.0)
    idx_ref[0] = jnp.where(inr, tgt, own)
    wx_ref[0] = fx * w
    wy_ref[0] = fy * w
    w_ref[0] = w


_prep = pl.pallas_call(
    _prep_body,
    grid=(B, H // RB),
    in_specs=[
        pl.BlockSpec((1, 2, RB, W), lambda b, r: (b, 0, r, 0)),
        pl.BlockSpec((1, 1, RB, W), lambda b, r: (b, 0, r, 0)),
    ],
    out_specs=[pl.BlockSpec((1, RB, W), lambda b, r: (b, r, 0))] * 4,
    out_shape=[
        jax.ShapeDtypeStruct((B, H, W), jnp.int32),
        jax.ShapeDtypeStruct((B, H, W), jnp.float32),
        jax.ShapeDtypeStruct((B, H, W), jnp.float32),
        jax.ShapeDtypeStruct((B, H, W), jnp.float32),
    ],
)


def _sc_body(idx_hbm, wx_hbm, wy_hbm, w_hbm, out_hbm,
             idx_v, vx_v, vy_v, vw_v, zb_v, acc_x, acc_y, acc_w):
    c = lax.axis_index("c")
    s = lax.axis_index("s")
    base = s * P

    def _zb(i, carry):
        zb_v[pl.ds(pl.multiple_of(i * 16, 16), 16)] = jnp.zeros((16,), jnp.float32)
        return carry

    lax.fori_loop(0, ZB // 16, _zb, 0)

    for k in range(BPC):
        b = c * BPC + k
        plsc.subcore_barrier()
        for acc in (acc_x, acc_y, acc_w):
            pltpu.sync_copy(zb_v, acc.at[pl.ds(base, ZB)])
            pltpu.sync_copy(zb_v, acc.at[pl.ds(base + ZB, ZB)])
        pltpu.sync_copy(idx_hbm.at[b, s], idx_v)
        pltpu.sync_copy(wx_hbm.at[b, s], vx_v)
        pltpu.sync_copy(wy_hbm.at[b, s], vy_v)
        pltpu.sync_copy(w_hbm.at[b, s], vw_v)
        plsc.subcore_barrier()

        def _scatter(j, carry):
            idx_row = idx_v.at[j]
            off = pl.ds(pl.multiple_of(j * CH, CH), CH)
            pltpu.sync_copy(vx_v.at[off], acc_x.at[idx_row], add=True)
            pltpu.sync_copy(vy_v.at[off], acc_y.at[idx_row], add=True)
            pltpu.sync_copy(vw_v.at[off], acc_w.at[idx_row], add=True)
            return carry

        lax.fori_loop(0, NCH, _scatter, 0)
        plsc.subcore_barrier()

        pltpu.sync_copy(acc_x.at[pl.ds(base, P)], vx_v)
        pltpu.sync_copy(acc_y.at[pl.ds(base, P)], vy_v)
        pltpu.sync_copy(acc_w.at[pl.ds(base, P)], vw_v)

        def _fin(i, carry):
            sl = pl.ds(pl.multiple_of(i * 16, 16), 16)
            ax = vx_v[sl]
            ay = vy_v[sl]
            aw = vw_v[sl]
            inv = jnp.where(ax != 0.0, 1.0 / (aw + 1e-7), 0.0)
            vx_v[sl] = ax * inv
            vy_v[sl] = ay * inv
            return carry

        lax.fori_loop(0, P // 16, _fin, 0)

        pltpu.sync_copy(vx_v, out_hbm.at[b, 0, pl.ds(base, P)])
        pltpu.sync_copy(vy_v, out_hbm.at[b, 1, pl.ds(base, P)])


def _build_scatter_norm():
    return pl.kernel(
        _sc_body,
        out_type=jax.ShapeDtypeStruct((B, 2, HW), jnp.float32),
        mesh=plsc.VectorSubcoreMesh(
            core_axis_name="c", subcore_axis_name="s", num_cores=NC, num_subcores=NS
        ),
        scratch_types=[
            pltpu.VMEM((NCH, CH), jnp.int32),
            pltpu.VMEM((P,), jnp.float32),
            pltpu.VMEM((P,), jnp.float32),
            pltpu.VMEM((P,), jnp.float32),
            pltpu.VMEM((ZB,), jnp.float32),
            pltpu.VMEM_SHARED((HW,), jnp.float32),
            pltpu.VMEM_SHARED((HW,), jnp.float32),
            pltpu.VMEM_SHARED((HW,), jnp.float32),
        ],
    )


def kernel(flow, inv_depth):
    idx, wx, wy, w = _prep(flow, inv_depth)
    out = _build_scatter_norm()(
        idx.reshape(B, NS, NCH, CH),
        wx.reshape(B, NS, P),
        wy.reshape(B, NS, P),
        w.reshape(B, NS, P),
    )
    return out.reshape(B, 2, H, W)


# speedup vs baseline: 45.6091x; 1.2620x over previous
import jax
import jax.numpy as jnp
from jax import lax
from jax.experimental import pallas as pl
from jax.experimental.pallas import tpu as pltpu
from jax.experimental.pallas import tpu_sc as plsc

B = 8
H = 512
W = 512
HW = H * W
NC = 2
NS = 16
P = HW // NS
CH = 128
NCH = P // CH
BPC = B // NC
ZB = 8192
SD = 4
RB = 256


def _prep_body(flow_ref, invd_ref, idx_ref, wx_ref, wy_ref, w_ref):
    r = pl.program_id(1)
    fx = flow_ref[0, 0]
    fy = flow_ref[0, 1]
    dv = invd_ref[0, 0]
    y = (lax.broadcasted_iota(jnp.int32, (RB, W), 0) + r * RB).astype(jnp.float32)
    x = lax.broadcasted_iota(jnp.int32, (RB, W), 1).astype(jnp.float32)
    tx = jnp.round(x - fx)
    ty = jnp.round(y - fy)
    inr = (tx >= 0.0) & (tx < float(W)) & (ty >= 0.0) & (ty < float(H))
    tgt = tx.astype(jnp.int32) + ty.astype(jnp.int32) * W
    own = (y * float(W) + x).astype(jnp.int32)
    w = jnp.where(inr, dv, 0.0)
    idx_ref[0] = jnp.where(inr, tgt, own)
    wx_ref[0] = fx * w
    wy_ref[0] = fy * w
    w_ref[0] = w


_prep = pl.pallas_call(
    _prep_body,
    grid=(B, H // RB),
    in_specs=[
        pl.BlockSpec((1, 2, RB, W), lambda b, r: (b, 0, r, 0)),
        pl.BlockSpec((1, 1, RB, W), lambda b, r: (b, 0, r, 0)),
    ],
    out_specs=[pl.BlockSpec((1, RB, W), lambda b, r: (b, r, 0))] * 4,
    out_shape=[
        jax.ShapeDtypeStruct((B, H, W), jnp.int32),
        jax.ShapeDtypeStruct((B, H, W), jnp.float32),
        jax.ShapeDtypeStruct((B, H, W), jnp.float32),
        jax.ShapeDtypeStruct((B, H, W), jnp.float32),
    ],
)


def _sc_body(idx_hbm, wx_hbm, wy_hbm, w_hbm, out_hbm,
             idx_v, vx_v, vy_v, vw_v, zb_v,
             acc_x, acc_y, acc_w):
    c = lax.axis_index("c")
    s = lax.axis_index("s")
    base = s * P

    def _zb(i, carry):
        zb_v[pl.ds(pl.multiple_of(i * 16, 16), 16)] = jnp.zeros((16,), jnp.float32)
        return carry

    lax.fori_loop(0, ZB // 16, _zb, 0, unroll=4)

    for k in range(BPC):
        b = c * BPC + k
        plsc.subcore_barrier()
        for acc in (acc_x, acc_y, acc_w):
            pltpu.sync_copy(zb_v, acc.at[pl.ds(base, ZB)])
            pltpu.sync_copy(zb_v, acc.at[pl.ds(base + ZB, ZB)])
        pltpu.sync_copy(idx_hbm.at[b, s], idx_v)
        pltpu.sync_copy(wx_hbm.at[b, s], vx_v)
        pltpu.sync_copy(wy_hbm.at[b, s], vy_v)
        pltpu.sync_copy(w_hbm.at[b, s], vw_v)
        plsc.subcore_barrier()

        pltpu.sync_copy(vx_v, acc_x.at[idx_v], add=True)
        pltpu.sync_copy(vy_v, acc_y.at[idx_v], add=True)
        pltpu.sync_copy(vw_v, acc_w.at[idx_v], add=True)
        plsc.subcore_barrier()

        pltpu.sync_copy(acc_x.at[pl.ds(base, P)], vx_v)
        pltpu.sync_copy(acc_y.at[pl.ds(base, P)], vy_v)
        pltpu.sync_copy(acc_w.at[pl.ds(base, P)], vw_v)

        def _fin(i, carry):
            sl = pl.ds(pl.multiple_of(i * 16, 16), 16)
            ax = vx_v[sl]
            ay = vy_v[sl]
            aw = vw_v[sl]
            inv = jnp.where(ax != 0.0, 1.0 / (aw + 1e-7), 0.0)
            vx_v[sl] = ax * inv
            vy_v[sl] = ay * inv
            return carry

        lax.fori_loop(0, P // 16, _fin, 0, unroll=4)

        pltpu.sync_copy(vx_v, out_hbm.at[b, 0, pl.ds(base, P)])
        pltpu.sync_copy(vy_v, out_hbm.at[b, 1, pl.ds(base, P)])


def _build_scatter_norm():
    return pl.kernel(
        _sc_body,
        out_type=jax.ShapeDtypeStruct((B, 2, HW), jnp.float32),
        mesh=plsc.VectorSubcoreMesh(
            core_axis_name="c", subcore_axis_name="s", num_cores=NC, num_subcores=NS
        ),
        scratch_types=[
            pltpu.VMEM((P,), jnp.int32),
            pltpu.VMEM((P,), jnp.float32),
            pltpu.VMEM((P,), jnp.float32),
            pltpu.VMEM((P,), jnp.float32),
            pltpu.VMEM((ZB,), jnp.float32),
            pltpu.VMEM_SHARED((HW,), jnp.float32),
            pltpu.VMEM_SHARED((HW,), jnp.float32),
            pltpu.VMEM_SHARED((HW,), jnp.float32),
        ],
    )


def kernel(flow, inv_depth):
    idx, wx, wy, w = _prep(flow, inv_depth)
    out = _build_scatter_norm()(
        idx.reshape(B, NS, P),
        wx.reshape(B, NS, P),
        wy.reshape(B, NS, P),
        w.reshape(B, NS, P),
    )
    return out.reshape(B, 2, H, W)
